# baseline (speedup 1.0000x reference)
import functools

import jax
import jax.numpy as jnp
from jax.experimental import pallas as pl
from jax.experimental.pallas import tpu as pltpu

B, N, D = 100, 1000, 128
_HI = jax.lax.Precision.HIGHEST


def _mm(a, b):
    return jax.lax.dot_general(a, b, (((1,), (0,)), ((), ())),
                               precision=_HI, preferred_element_type=jnp.float32)


def _gru(x, agg, WihT, WhhT, bih, bhh):
    gi = _mm(agg, WihT) + bih
    gh = _mm(x, WhhT) + bhh
    i_r, i_z, i_n = gi[:, :D], gi[:, D:2 * D], gi[:, 2 * D:]
    h_r, h_z, h_n = gh[:, :D], gh[:, D:2 * D], gh[:, 2 * D:]
    r = jax.nn.sigmoid(i_r + h_r)
    z = jax.nn.sigmoid(i_z + h_z)
    n = jnp.tanh(i_n + r * h_n)
    return (1.0 - z) * n + z * x


def _tour_kernel(x_ref, src_row_ref, srcT_ref, dst_row_ref, dstT_ref, batch_ref,
                 W0_ref, WihT0_ref, WhhT0_ref, bih0_ref, bhh0_ref,
                 W1_ref, WihT1_ref, WhhT1_ref, bih1_ref, bhh1_ref,
                 gnw_ref, gnb_ref, gnm_ref,
                 out_ref, S_acc, Q_acc, C_acc):
    g = pl.program_id(0)

    x = x_ref[0]
    src_row = src_row_ref[0]
    dst_row = dst_row_ref[0]
    srcT = srcT_ref[0]
    dstT = dstT_ref[0]

    lane = jax.lax.broadcasted_iota(jnp.int32, (1, N), 1)
    sub = jax.lax.broadcasted_iota(jnp.int32, (N, 1), 0)

    def layer(_, carry):
        x0, x1 = carry
        m0 = _mm(x0, W0_ref[...])
        gath0 = _mm((srcT == lane).astype(jnp.float32), m0)
        agg0 = _mm((sub == dst_row).astype(jnp.float32), gath0)
        x0 = _gru(x0, agg0, WihT0_ref[...], WhhT0_ref[...], bih0_ref[...], bhh0_ref[...])
        m1 = _mm(x1, W1_ref[...])
        gath1 = _mm((dstT == lane).astype(jnp.float32), m1)
        agg1 = _mm((sub == src_row).astype(jnp.float32), gath1)
        x1 = _gru(x1, agg1, WihT1_ref[...], WhhT1_ref[...], bih1_ref[...], bhh1_ref[...])
        return x0, x1

    x0, x1 = jax.lax.fori_loop(0, 3, layer, (x, x))

    xs = x0 + x1
    xf = 0.5 * xs * (1.0 + jax.lax.erf(xs * 0.7071067811865476))

    bids = jax.lax.broadcasted_iota(jnp.int32, (B, 1), 0)
    BOH = (bids == batch_ref[0]).astype(jnp.float32)
    pS = _mm(BOH, xf)
    pQ = _mm(BOH, xf * xf)
    pC = jnp.sum(BOH, axis=1, keepdims=True)

    @pl.when(g == 0)
    def _():
        S_acc[...] = pS
        Q_acc[...] = pQ
        C_acc[...] = pC

    @pl.when(g > 0)
    def _():
        S_acc[...] += pS
        Q_acc[...] += pQ
        C_acc[...] += pC

    @pl.when(g == B - 1)
    def _():
        S = S_acc[...]
        Q = Q_acc[...]
        c = jnp.maximum(C_acc[...], 1.0)
        ms = gnm_ref[...]
        mean = S / c
        sum_out = S - c * (mean * ms)
        var = (Q - 2.0 * (mean * ms) * S + c * (mean * ms) ** 2) / c
        std = jnp.sqrt(var + 1e-5)
        out_ref[...] = gnw_ref[...] * sum_out / std + c * gnb_ref[...]


@functools.partial(jax.jit, static_argnames=())
def kernel(dense_x, dense_edge_index, batch, W0, Wih0, Whh0, bih0, bhh0,
           W1, Wih1, Whh1, bih1, bhh1, gn_weight, gn_bias, gn_mean_scale):
    src = dense_edge_index[:, :, 0]
    dst = dense_edge_index[:, :, 1]
    src_row = src.reshape(B, 1, N)
    dst_row = dst.reshape(B, 1, N)
    srcT = src.reshape(B, N, 1)
    dstT = dst.reshape(B, N, 1)
    batch_row = batch.reshape(B, 1, N)

    full = lambda shape: pl.BlockSpec(shape, lambda g: (0,) * len(shape))
    per_g = lambda shape: pl.BlockSpec((1,) + shape, lambda g: (g,) + (0,) * len(shape))

    args = (
        dense_x, src_row, srcT, dst_row, dstT, batch_row,
        W0, Wih0.T, Whh0.T, bih0.reshape(1, 3 * D), bhh0.reshape(1, 3 * D),
        W1, Wih1.T, Whh1.T, bih1.reshape(1, 3 * D), bhh1.reshape(1, 3 * D),
        gn_weight.reshape(1, D), gn_bias.reshape(1, D), gn_mean_scale.reshape(1, D),
    )
    in_specs = [
        per_g((N, D)), per_g((1, N)), per_g((N, 1)), per_g((1, N)), per_g((N, 1)),
        per_g((1, N)),
        full((D, D)), full((D, 3 * D)), full((D, 3 * D)), full((1, 3 * D)), full((1, 3 * D)),
        full((D, D)), full((D, 3 * D)), full((D, 3 * D)), full((1, 3 * D)), full((1, 3 * D)),
        full((1, D)), full((1, D)), full((1, D)),
    ]

    return pl.pallas_call(
        _tour_kernel,
        grid=(B,),
        in_specs=in_specs,
        out_specs=pl.BlockSpec((B, D), lambda g: (0, 0)),
        out_shape=jax.ShapeDtypeStruct((B, D), jnp.float32),
        scratch_shapes=[
            pltpu.VMEM((B, D), jnp.float32),
            pltpu.VMEM((B, D), jnp.float32),
            pltpu.VMEM((B, 1), jnp.float32),
        ],
        compiler_params=pltpu.CompilerParams(vmem_limit_bytes=120 * 1024 * 1024),
    )(*args)
